# baseline (speedup 1.0000x reference)
import functools

import jax
import jax.numpy as jnp
from jax import lax
from jax.experimental import pallas as pl
from jax.experimental.pallas import tpu as pltpu
from jax.experimental.pallas import tpu_sc as plsc

N_TOK = 2048
D_MODEL = 768
GENRE_DIM = 64
N_EXPERTS = 8
D_FF = 3072
TOP_K = 2

BLK = 128
N_BLOCKS = 40
NP = N_BLOCKS * BLK
FF_CHUNK = 768
N_FF = D_FF // FF_CHUNK

NC, NS = 2, 16
NW = NC * NS
TPW = N_TOK // NW


def _lane_shift_exscan(v):
    s = jnp.concatenate([jnp.zeros((1, 1), v.dtype), v[:, : N_EXPERTS - 1]], axis=1)
    for sh in (1, 2, 4):
        s = s + jnp.concatenate(
            [jnp.zeros((1, sh), v.dtype), s[:, : N_EXPERTS - sh]], axis=1)
    return s


def _gate_body(x_ref, g_ref, wg_ref, bg_ref,
               pos0_ref, pos1_ref, w0_ref, w1_ref, be_ref):
    gate_in = jnp.concatenate([x_ref[...], g_ref[...]], axis=1)
    logits = jnp.dot(gate_in, wg_ref[...], preferred_element_type=jnp.float32)
    logits = logits + bg_ref[...]

    e_iota = lax.broadcasted_iota(jnp.int32, (N_TOK, N_EXPERTS), 1)
    m0 = jnp.max(logits, axis=1, keepdims=True)
    i0 = jnp.min(jnp.where(logits == m0, e_iota, N_EXPERTS), axis=1, keepdims=True)
    masked = jnp.where(e_iota == i0, -jnp.inf, logits)
    m1 = jnp.max(masked, axis=1, keepdims=True)
    i1 = jnp.min(jnp.where(masked == m1, e_iota, N_EXPERTS), axis=1, keepdims=True)

    d = jnp.exp(m1 - m0)
    w0_ref[...] = 1.0 / (1.0 + d)
    w1_ref[...] = d / (1.0 + d)

    onehot0 = (e_iota == i0).astype(jnp.float32)
    onehot1 = (e_iota == i1).astype(jnp.float32)

    oh = jnp.concatenate([onehot0, onehot1], axis=1)
    s = jnp.concatenate(
        [jnp.zeros((1, 2 * N_EXPERTS), jnp.float32), oh[: N_TOK - 1]], axis=0)
    sh = 1
    while sh < N_TOK:
        s = s + jnp.concatenate(
            [jnp.zeros((sh, 2 * N_EXPERTS), jnp.float32), s[: N_TOK - sh]], axis=0)
        sh *= 2
    csum0, csum1 = s[:, :N_EXPERTS], s[:, N_EXPERTS:]

    cnt0 = jnp.sum(onehot0, axis=0, keepdims=True)
    cnt1 = jnp.sum(onehot1, axis=0, keepdims=True)
    ct = (cnt0 + cnt1).astype(jnp.int32)
    padded = ((ct + (BLK - 1)) >> 7) << 7
    poff = _lane_shift_exscan(padded)

    poff_f = poff.astype(jnp.float32)
    poff_b = jnp.broadcast_to(poff_f, (N_TOK, N_EXPERTS))
    cnt0_b = jnp.broadcast_to(cnt0, (N_TOK, N_EXPERTS))
    rank0 = jnp.sum(csum0 * onehot0, axis=1, keepdims=True)
    rank1 = jnp.sum(csum1 * onehot1, axis=1, keepdims=True)
    base0 = jnp.sum(poff_b * onehot0, axis=1, keepdims=True)
    base1 = jnp.sum((poff_b + cnt0_b) * onehot1, axis=1, keepdims=True)
    pos0_ref[...] = (base0 + rank0).astype(jnp.int32)
    pos1_ref[...] = (base1 + rank1).astype(jnp.int32)

    b_iota = lax.broadcasted_iota(jnp.int32, (1, 128), 1) * BLK
    be = jnp.full((1, 128), -1, jnp.int32)
    for e in range(N_EXPERTS):
        start_e = jnp.broadcast_to(poff[:, e : e + 1], (1, 128))
        be = be + (b_iota >= start_e).astype(jnp.int32)
    be_ref[...] = be


def _ffn_body(be_ref, xs_ref, w1_ref, b1_ref, w2_ref, b2_ref, ys_ref):
    del be_ref
    j = pl.program_id(1)
    h = jnp.dot(xs_ref[...], w1_ref[0], preferred_element_type=jnp.float32)
    h = jax.nn.gelu(h + b1_ref[0])
    y = jnp.dot(h, w2_ref[0], preferred_element_type=jnp.float32)

    @pl.when(j == 0)
    def _():
        ys_ref[...] = jnp.broadcast_to(b2_ref[0], ys_ref.shape)

    ys_ref[...] += y


def _combine_body(g0_ref, g1_ref, w0_ref, w1_ref, out_ref):
    out_ref[...] = w0_ref[...] * g0_ref[...] + w1_ref[...] * g1_ref[...]


@functools.cache
def _sc_kernels():
    mesh = plsc.VectorSubcoreMesh(core_axis_name="c", subcore_axis_name="s")

    @functools.partial(
        pl.kernel, mesh=mesh,
        out_type=jax.ShapeDtypeStruct((NP, D_MODEL), jnp.float32),
        scratch_types=[
            pltpu.VMEM((TPW, D_MODEL), jnp.float32),
            pltpu.VMEM((TPW,), jnp.int32),
            pltpu.VMEM((TPW,), jnp.int32),
            pltpu.SemaphoreType.DMA,
        ],
    )
    def scatter_rows(x_hbm, p0_hbm, p1_hbm, xs_hbm, xv, i0v, i1v, sem):
        wid = lax.axis_index("s") * NC + lax.axis_index("c")
        base = wid * TPW
        pltpu.sync_copy(x_hbm.at[pl.ds(base, TPW)], xv)
        pltpu.sync_copy(p0_hbm.at[pl.ds(base, TPW)], i0v)
        pltpu.sync_copy(p1_hbm.at[pl.ds(base, TPW)], i1v)
        pltpu.async_copy(xv, xs_hbm.at[i0v], sem).wait()
        pltpu.async_copy(xv, xs_hbm.at[i1v], sem).wait()

    @functools.partial(
        pl.kernel, mesh=mesh,
        out_type=(jax.ShapeDtypeStruct((N_TOK, D_MODEL), jnp.float32),
                  jax.ShapeDtypeStruct((N_TOK, D_MODEL), jnp.float32)),
        scratch_types=[
            pltpu.VMEM((TPW, D_MODEL), jnp.float32),
            pltpu.VMEM((TPW,), jnp.int32),
            pltpu.SemaphoreType.DMA,
        ],
    )
    def gather_rows(ys_hbm, p0_hbm, p1_hbm, g0_hbm, g1_hbm, gv, iv, sem):
        wid = lax.axis_index("s") * NC + lax.axis_index("c")
        base = wid * TPW
        pltpu.sync_copy(p0_hbm.at[pl.ds(base, TPW)], iv)
        pltpu.async_copy(ys_hbm.at[iv], gv, sem).wait()
        pltpu.sync_copy(gv, g0_hbm.at[pl.ds(base, TPW)])
        pltpu.sync_copy(p1_hbm.at[pl.ds(base, TPW)], iv)
        pltpu.async_copy(ys_hbm.at[iv], gv, sem).wait()
        pltpu.sync_copy(gv, g1_hbm.at[pl.ds(base, TPW)])

    return scatter_rows, gather_rows


_gate_call = pl.pallas_call(
    _gate_body,
    out_shape=(
        jax.ShapeDtypeStruct((N_TOK, 1), jnp.int32),
        jax.ShapeDtypeStruct((N_TOK, 1), jnp.int32),
        jax.ShapeDtypeStruct((N_TOK, 1), jnp.float32),
        jax.ShapeDtypeStruct((N_TOK, 1), jnp.float32),
        jax.ShapeDtypeStruct((1, 128), jnp.int32),
    ),
)

_ffn_call = pl.pallas_call(
    _ffn_body,
    grid_spec=pltpu.PrefetchScalarGridSpec(
        num_scalar_prefetch=1,
        grid=(N_BLOCKS, N_FF),
        in_specs=[
            pl.BlockSpec((BLK, D_MODEL), lambda b, j, be: (b, 0)),
            pl.BlockSpec((1, D_MODEL, FF_CHUNK), lambda b, j, be: (be[b], 0, j)),
            pl.BlockSpec((1, 1, FF_CHUNK), lambda b, j, be: (be[b], 0, j)),
            pl.BlockSpec((1, FF_CHUNK, D_MODEL), lambda b, j, be: (be[b], j, 0)),
            pl.BlockSpec((1, 1, D_MODEL), lambda b, j, be: (be[b], 0, 0)),
        ],
        out_specs=pl.BlockSpec((BLK, D_MODEL), lambda b, j, be: (b, 0)),
    ),
    out_shape=jax.ShapeDtypeStruct((NP, D_MODEL), jnp.float32),
    compiler_params=pltpu.CompilerParams(
        dimension_semantics=("arbitrary", "arbitrary")),
)

_combine_call = pl.pallas_call(
    _combine_body,
    out_shape=jax.ShapeDtypeStruct((N_TOK, D_MODEL), jnp.float32),
)


def kernel(x, genre_emb, Wg, bg, W1, b1, W2, b2):
    pos0_2d, pos1_2d, w0, w1, be = _gate_call(
        x, genre_emb, Wg, bg.reshape(1, N_EXPERTS))
    pos0 = pos0_2d.reshape(N_TOK)
    pos1 = pos1_2d.reshape(N_TOK)
    be40 = be.reshape(128)[:N_BLOCKS]
    scatter_rows, gather_rows = _sc_kernels()
    xs = scatter_rows(x, pos0, pos1)
    ys = _ffn_call(be40, xs, W1,
                   b1.reshape(N_EXPERTS, 1, D_FF), W2,
                   b2.reshape(N_EXPERTS, 1, D_MODEL))
    g0, g1 = gather_rows(ys, pos0, pos1)
    return _combine_call(g0, g1, w0, w1)
